# baseline (speedup 1.0000x reference)
import math
import functools

import jax
import jax.numpy as jnp
from jax.experimental import pallas as pl
from jax.experimental.pallas import tpu as pltpu


def _round_up(x, m):
    return (x + m - 1) // m * m


def _gather_scale_kernel(ids_ref, table_hbm, out_ref, sems, *, tile, scale,
                         n_sub):
    V = table_hbm.shape[0]
    base = pl.program_id(0) * tile
    sub = tile // n_sub

    def issue(t, sem, prio):
        row = ids_ref[base + t]
        row = jnp.minimum(jnp.maximum(row, 0), V - 1)
        pltpu.async_copy(
            table_hbm.at[pl.ds(row, 1), :],
            out_ref.at[pl.ds(t, 1), :],
            sem,
            priority=prio,
        )

    for j in range(n_sub):
        @pl.loop(j * sub // 2, (j + 1) * sub // 2)
        def _(tq, j=j):
            for u in range(2):
                issue(tq * 2 + u, sems.at[j], u)

    for j in range(n_sub):
        pltpu.make_async_copy(
            table_hbm.at[pl.ds(0, sub), :],
            out_ref.at[pl.ds(j * sub, sub), :],
            sems.at[j],
        ).wait()
        out_ref[pl.ds(j * sub, sub), :] = (
            out_ref[pl.ds(j * sub, sub), :] * jnp.float32(scale))


def kernel(x_ids, table):
    B, S = x_ids.shape
    V, D = table.shape
    N = B * S
    scale = math.sqrt(D)

    tile = min(2048, _round_up(N, 8))
    if _round_up(N, tile) // tile < 2 and N > 8:
        tile = min(tile, _round_up((N + 1) // 2, 8))
    n_pad = _round_up(N, tile)
    n_sub = 4 if tile % 16 == 0 else 2

    flat_ids = x_ids.reshape(N).astype(jnp.int32)
    if n_pad != N:
        flat_ids = jnp.pad(flat_ids, (0, n_pad - N))

    itemsize = jnp.dtype(table.dtype).itemsize
    vmem_limit = int(min(4 * tile * D * itemsize + (8 << 20), 56 << 20))

    grid_spec = pltpu.PrefetchScalarGridSpec(
        num_scalar_prefetch=1,
        grid=(n_pad // tile,),
        in_specs=[pl.BlockSpec(memory_space=pl.ANY)],
        out_specs=pl.BlockSpec((tile, D), lambda i, ids: (i, 0)),
        scratch_shapes=[pltpu.SemaphoreType.DMA((n_sub,))],
    )
    out_flat = pl.pallas_call(
        functools.partial(_gather_scale_kernel, tile=tile, scale=scale,
                          n_sub=n_sub),
        out_shape=jax.ShapeDtypeStruct((n_pad, D), table.dtype),
        grid_spec=grid_spec,
        compiler_params=pltpu.CompilerParams(
            dimension_semantics=("parallel",),
            vmem_limit_bytes=vmem_limit,
            disable_bounds_checks=True,
        ),
        name="embedding_gather_scale",
    )(flat_ids, table)

    return out_flat[:N].reshape(B, S, D)
